# Initial kernel scaffold; baseline (speedup 1.0000x reference)
import jax
import jax.numpy as jnp
from jax.experimental import pallas as pl


def kernel(query, value, reference_points, spatial_shapes, W_off, b_off, W_attn, b_attn, W_val, b_val, W_out, b_out):
    raise NotImplementedError("write your pallas kernel here")


# speedup vs baseline: 1.4350x; 1.4350x over previous
import functools

import jax
import jax.numpy as jnp
from jax import lax
from jax.experimental import pallas as pl
from jax.experimental.pallas import tpu as pltpu
from jax.experimental.pallas import tpu_sc as plsc

EMBED = 256
NH = 8
NP = 4
H_ = 100
W_ = 100
HW = H_ * W_
BS = 4
NQ = 900
HD = EMBED // NH
NTERM = NP * 4
NW = 32
CH = 60
NCHUNK = NQ // CH
GJ = 8
GB = CH * NTERM // GJ

POS_TILE = 1000


def _value_proj_body(v_ref, wt_ref, b_ref, out_ref):
    acc = jnp.dot(v_ref[0], wt_ref[...], preferred_element_type=jnp.float32)
    acc = acc + b_ref[...][None, :]
    for h in range(NH):
        out_ref[0, h] = acc[:, h * HD:(h + 1) * HD]


def _value_proj(value, w_val_t, b_val):
    return pl.pallas_call(
        _value_proj_body,
        grid=(BS, HW // POS_TILE),
        in_specs=[
            pl.BlockSpec((1, POS_TILE, EMBED), lambda b, t: (b, t, 0)),
            pl.BlockSpec((EMBED, EMBED), lambda b, t: (0, 0)),
            pl.BlockSpec((EMBED,), lambda b, t: (0,)),
        ],
        out_specs=pl.BlockSpec((1, NH, POS_TILE, HD), lambda b, t: (b, 0, t, 0)),
        out_shape=jax.ShapeDtypeStruct((BS, NH, HW, HD), jnp.float32),
    )(value, w_val_t, b_val)


def _query_side_body(q_ref, r_ref, wx_ref, bx_ref, wy_ref, by_ref,
                     wa_ref, ba_ref, g_ref, idx_ref, wt_ref):
    b = pl.program_id(0)
    q = q_ref[0]
    refx = r_ref[0, :, 0:1]
    refy = r_ref[0, :, 1:2]

    px = jnp.dot(q, wx_ref[...].T, preferred_element_type=jnp.float32)
    px = px + bx_ref[...][None, :] + (refx * W_ - 0.5)
    py = jnp.dot(q, wy_ref[...].T, preferred_element_type=jnp.float32)
    py = py + by_ref[...][None, :] + (refy * H_ - 0.5)

    logits = jnp.dot(q, wa_ref[...].T, preferred_element_type=jnp.float32)
    logits = logits + ba_ref[...][None, :]
    m = jnp.max(logits, axis=1, keepdims=True)
    e = jnp.exp(logits - m)
    s = jnp.dot(e, g_ref[...], preferred_element_type=jnp.float32)
    aw = 4.0 * e / s

    col = lax.broadcasted_iota(jnp.int32, (NQ, NH * NTERM), 1)
    c = col % 4
    dx = (c % 2).astype(jnp.float32)
    dy = (c // 2).astype(jnp.float32)

    x0 = jnp.floor(px)
    y0 = jnp.floor(py)
    fx = px - x0
    fy = py - y0
    ixf = x0 + dx
    iyf = y0 + dy
    valid = ((ixf >= 0.0) & (ixf <= W_ - 1.0)
             & (iyf >= 0.0) & (iyf <= H_ - 1.0)).astype(jnp.float32)
    wx = dx * fx + (1.0 - dx) * (1.0 - fx)
    wy = dy * fy + (1.0 - dy) * (1.0 - fy)
    wt_ref[0] = aw * wx * wy * valid

    ixc = jnp.clip(ixf, 0.0, W_ - 1.0).astype(jnp.int32)
    iyc = jnp.clip(iyf, 0.0, H_ - 1.0).astype(jnp.int32)
    head = col // NTERM
    base = (b * NH + head) * HW
    idx_ref[0] = base + iyc * W_ + ixc


def _query_side(query, ref_pts, wx, bx, wy, by, wa, ba, g):
    return pl.pallas_call(
        _query_side_body,
        grid=(BS,),
        in_specs=[
            pl.BlockSpec((1, NQ, EMBED), lambda b: (b, 0, 0)),
            pl.BlockSpec((1, NQ, 2), lambda b: (b, 0, 0)),
            pl.BlockSpec((NH * NTERM, EMBED), lambda b: (0, 0)),
            pl.BlockSpec((NH * NTERM,), lambda b: (0,)),
            pl.BlockSpec((NH * NTERM, EMBED), lambda b: (0, 0)),
            pl.BlockSpec((NH * NTERM,), lambda b: (0,)),
            pl.BlockSpec((NH * NTERM, EMBED), lambda b: (0, 0)),
            pl.BlockSpec((NH * NTERM,), lambda b: (0,)),
            pl.BlockSpec((NH * NTERM, NH * NTERM), lambda b: (0, 0)),
        ],
        out_specs=[
            pl.BlockSpec((1, NQ, NH * NTERM), lambda b: (b, 0, 0)),
            pl.BlockSpec((1, NQ, NH * NTERM), lambda b: (b, 0, 0)),
        ],
        out_shape=[
            jax.ShapeDtypeStruct((BS, NQ, NH * NTERM), jnp.int32),
            jax.ShapeDtypeStruct((BS, NQ, NH * NTERM), jnp.float32),
        ],
    )(query, ref_pts, wx, bx, wy, by, wa, ba, g)


def _sc_gather_reduce(vt_flat, idxr, wtr):
    mesh = plsc.VectorSubcoreMesh(core_axis_name="c", subcore_axis_name="s")

    @functools.partial(
        pl.kernel,
        mesh=mesh,
        compiler_params=pltpu.CompilerParams(use_tc_tiling_on_sc=False),
        out_type=jax.ShapeDtypeStruct((NW, NCHUNK, CH, HD), jnp.float32),
        scratch_types=[
            pltpu.VMEM((GJ, GB), jnp.int32),
            pltpu.VMEM((CH * NTERM, HD), jnp.float32),
            pltpu.VMEM((CH, NTERM), jnp.float32),
            pltpu.VMEM((CH, HD), jnp.float32),
            pltpu.SemaphoreType.DMA,
        ],
    )
    def body(vt_hbm, idx_hbm, wt_hbm, out_hbm, idx_v, rows_v, wt_v, out_v, sem):
        w = lax.axis_index("s") * 2 + lax.axis_index("c")

        def chunk(k, carry):
            pltpu.sync_copy(idx_hbm.at[w, k], idx_v)
            pltpu.sync_copy(wt_hbm.at[w, k], wt_v)
            handles = []
            for j in range(GJ):
                handles.append(pltpu.async_copy(
                    vt_hbm.at[idx_v.at[j]],
                    rows_v.at[pl.ds(j * GB, GB)], sem))
            for hnd in handles:
                hnd.wait()

            def qstep(qi, c2):
                t0 = qi * NTERM
                wrow = wt_v[qi, :]
                acc0 = jnp.zeros((16,), jnp.float32)
                acc1 = jnp.zeros((16,), jnp.float32)
                for j in range(NTERM):
                    wsc = wrow[j]
                    acc0 = acc0 + wsc * rows_v[t0 + j, pl.ds(0, 16)]
                    acc1 = acc1 + wsc * rows_v[t0 + j, pl.ds(16, 16)]
                out_v[qi, pl.ds(0, 16)] = acc0
                out_v[qi, pl.ds(16, 16)] = acc1
                return c2

            lax.fori_loop(0, CH, qstep, 0)
            pltpu.sync_copy(out_v, out_hbm.at[w, k])
            return carry

        lax.fori_loop(0, NCHUNK, chunk, 0)

    return body(vt_flat, idxr, wtr)


def _out_proj_body(s_ref, wt_ref, b_ref, o_ref):
    acc = jnp.dot(s_ref[0], wt_ref[...], preferred_element_type=jnp.float32)
    o_ref[0] = acc + b_ref[...][None, :]


def _out_proj(sampled, w_out_t, b_out):
    return pl.pallas_call(
        _out_proj_body,
        grid=(BS,),
        in_specs=[
            pl.BlockSpec((1, NQ, EMBED), lambda b: (b, 0, 0)),
            pl.BlockSpec((EMBED, EMBED), lambda b: (0, 0)),
            pl.BlockSpec((EMBED,), lambda b: (0,)),
        ],
        out_specs=pl.BlockSpec((1, NQ, EMBED), lambda b: (b, 0, 0)),
        out_shape=jax.ShapeDtypeStruct((BS, NQ, EMBED), jnp.float32),
    )(sampled, w_out_t, b_out)


def kernel(query, value, reference_points, spatial_shapes,
           W_off, b_off, W_attn, b_attn, W_val, b_val, W_out, b_out):
    f32 = jnp.float32
    j = jnp.arange(NH * NTERM)
    head = j // NTERM
    p = (j % NTERM) // 4
    kx = (head * NP + p) * 2
    ka = head * NP + p
    wx = W_off[kx].astype(f32)
    bx = b_off[kx].astype(f32)
    wy = W_off[kx + 1].astype(f32)
    by = b_off[kx + 1].astype(f32)
    wa = W_attn[ka].astype(f32)
    ba = b_attn[ka].astype(f32)
    g = (head[:, None] == head[None, :]).astype(f32)

    vt = _value_proj(value, W_val.T.astype(f32), b_val.astype(f32))
    vt_flat = vt.reshape(BS * NH * HW, HD)

    ref_pts = reference_points[:, :, 0, :]
    idx, wt = _query_side(query, ref_pts, wx, bx, wy, by, wa, ba, g)

    idxr = (idx.reshape(BS, NQ, NH, NTERM).transpose(0, 2, 1, 3)
            .reshape(NW, NCHUNK, GJ, GB))
    wtr = (wt.reshape(BS, NQ, NH, NTERM).transpose(0, 2, 1, 3)
           .reshape(NW, NCHUNK, CH, NTERM))

    sc = _sc_gather_reduce(vt_flat, idxr, wtr)
    sampled = (sc.reshape(BS, NH, NQ, HD).transpose(0, 2, 1, 3)
               .reshape(BS, NQ, EMBED))

    return _out_proj(sampled, W_out.T.astype(f32), b_out.astype(f32))


# speedup vs baseline: 1.5956x; 1.1119x over previous
import functools

import jax
import jax.numpy as jnp
from jax import lax
from jax.experimental import pallas as pl
from jax.experimental.pallas import tpu as pltpu
from jax.experimental.pallas import tpu_sc as plsc

EMBED = 256
NH = 8
NP = 4
H_ = 100
W_ = 100
HW = H_ * W_
BS = 4
NQ = 900
HD = EMBED // NH
PATCH = 4 * HD
NW = 32
CH = 60
NCHUNK = NQ // CH
GJ = 2
GB = CH * NP // GJ

POS_TILE = 1000
HALO = 104


def _patch_table_body(v_ref, vn_ref, wt_ref, b_ref, out_ref):
    acc = jnp.dot(v_ref[0], wt_ref[...], preferred_element_type=jnp.float32)
    acc = acc + b_ref[...][None, :]
    halo = jnp.dot(vn_ref[0, :HALO], wt_ref[...],
                   preferred_element_type=jnp.float32)
    halo = halo + b_ref[...][None, :]
    ext = jnp.concatenate([acc, halo], axis=0)
    s0 = ext[0:POS_TILE]
    s1 = ext[1:POS_TILE + 1]
    s2 = ext[W_:POS_TILE + W_]
    s3 = ext[W_ + 1:POS_TILE + W_ + 1]
    for h in range(NH):
        sl = slice(h * HD, (h + 1) * HD)
        out_ref[h] = jnp.concatenate(
            [s0[:, sl], s1[:, sl], s2[:, sl], s3[:, sl]], axis=1)


def _patch_table(value, w_val_t, b_val):
    nt = HW // POS_TILE
    return pl.pallas_call(
        _patch_table_body,
        grid=(BS, nt),
        in_specs=[
            pl.BlockSpec((1, POS_TILE, EMBED), lambda b, t: (b, t, 0)),
            pl.BlockSpec((1, POS_TILE, EMBED),
                         lambda b, t: (b, jnp.minimum(t + 1, nt - 1), 0)),
            pl.BlockSpec((EMBED, EMBED), lambda b, t: (0, 0)),
            pl.BlockSpec((EMBED,), lambda b, t: (0,)),
        ],
        out_specs=pl.BlockSpec((NH, POS_TILE, PATCH), lambda b, t: (b, t, 0)),
        out_shape=jax.ShapeDtypeStruct((BS * NH, HW, PATCH), jnp.float32),
    )(value, value, w_val_t, b_val)


def _query_side_body(q_ref, r_ref, wx_ref, bx_ref, wy_ref, by_ref,
                     wa_ref, ba_ref, g_ref, idx_ref, wt_ref):
    b = pl.program_id(0)
    q = q_ref[0]
    refx = r_ref[0, :, 0:1]
    refy = r_ref[0, :, 1:2]

    px = jnp.dot(q, wx_ref[...].T, preferred_element_type=jnp.float32)
    px = px + bx_ref[...][None, :] + (refx * W_ - 0.5)
    py = jnp.dot(q, wy_ref[...].T, preferred_element_type=jnp.float32)
    py = py + by_ref[...][None, :] + (refy * H_ - 0.5)

    x0 = jnp.clip(jnp.floor(px), 0.0, W_ - 2.0)
    y0 = jnp.clip(jnp.floor(py), 0.0, H_ - 2.0)

    head32 = lax.broadcasted_iota(jnp.int32, (NQ, NH * NP), 1) // NP
    base = (b * NH + head32) * HW
    idx_ref[0] = base + y0.astype(jnp.int32) * W_ + x0.astype(jnp.int32)

    logits = jnp.dot(q, wa_ref[...].T, preferred_element_type=jnp.float32)
    logits = logits + ba_ref[...][None, :]
    m = jnp.max(logits, axis=1, keepdims=True)
    e = jnp.exp(logits - m)
    s = jnp.dot(e, g_ref[...], preferred_element_type=jnp.float32)
    aw = e / s

    parts = []
    for sub in range(4):
        sx = float(sub % 2)
        sy = float(sub // 2)
        tx = jnp.maximum(0.0, 1.0 - jnp.abs(px - (x0 + sx)))
        ty = jnp.maximum(0.0, 1.0 - jnp.abs(py - (y0 + sy)))
        parts.append(aw * tx * ty)
    wt_ref[0] = jnp.concatenate(parts, axis=1)


def _query_side(query, ref_pts, wx, bx, wy, by, wa, ba, g):
    n128 = NH * NP * 4
    return pl.pallas_call(
        _query_side_body,
        grid=(BS,),
        in_specs=[
            pl.BlockSpec((1, NQ, EMBED), lambda b: (b, 0, 0)),
            pl.BlockSpec((1, NQ, 2), lambda b: (b, 0, 0)),
            pl.BlockSpec((NH * NP, EMBED), lambda b: (0, 0)),
            pl.BlockSpec((NH * NP,), lambda b: (0,)),
            pl.BlockSpec((NH * NP, EMBED), lambda b: (0, 0)),
            pl.BlockSpec((NH * NP,), lambda b: (0,)),
            pl.BlockSpec((NH * NP, EMBED), lambda b: (0, 0)),
            pl.BlockSpec((NH * NP,), lambda b: (0,)),
            pl.BlockSpec((NH * NP, NH * NP), lambda b: (0, 0)),
        ],
        out_specs=[
            pl.BlockSpec((1, NQ, NH * NP), lambda b: (b, 0, 0)),
            pl.BlockSpec((1, NQ, n128), lambda b: (b, 0, 0)),
        ],
        out_shape=[
            jax.ShapeDtypeStruct((BS, NQ, NH * NP), jnp.int32),
            jax.ShapeDtypeStruct((BS, NQ, n128), jnp.float32),
        ],
    )(query, ref_pts, wx, bx, wy, by, wa, ba, g)


def _sc_gather_reduce(vt_flat, idxr, wtr):
    mesh = plsc.VectorSubcoreMesh(core_axis_name="c", subcore_axis_name="s")

    @functools.partial(
        pl.kernel,
        mesh=mesh,
        compiler_params=pltpu.CompilerParams(use_tc_tiling_on_sc=False),
        out_type=jax.ShapeDtypeStruct((NW, NCHUNK, CH, HD), jnp.float32),
        scratch_types=[
            pltpu.VMEM((GJ, GB), jnp.int32),
            pltpu.VMEM((CH * NP, PATCH), jnp.float32),
            pltpu.VMEM((CH, NP * 4), jnp.float32),
            pltpu.VMEM((CH, HD), jnp.float32),
            pltpu.SemaphoreType.DMA,
        ],
    )
    def body(vt_hbm, idx_hbm, wt_hbm, out_hbm, idx_v, rows_v, wt_v, out_v, sem):
        w = lax.axis_index("s") * 2 + lax.axis_index("c")
        b = w // NH
        h = w % NH

        def chunk(k, carry):
            pltpu.sync_copy(idx_hbm.at[w, k], idx_v)
            pltpu.sync_copy(wt_hbm.at[w, k], wt_v)
            handles = []
            for j in range(GJ):
                handles.append(pltpu.async_copy(
                    vt_hbm.at[idx_v.at[j]],
                    rows_v.at[pl.ds(j * GB, GB)], sem))
            for hnd in handles:
                hnd.wait()

            def qstep(qi, c2):
                wrow = wt_v[qi, :]
                acc0 = jnp.zeros((16,), jnp.float32)
                acc1 = jnp.zeros((16,), jnp.float32)
                for p in range(NP):
                    r = qi * NP + p
                    for s in range(4):
                        wsc = wrow[s * 4 + p]
                        acc0 = acc0 + wsc * rows_v[r, pl.ds(s * HD, 16)]
                        acc1 = acc1 + wsc * rows_v[r, pl.ds(s * HD + 16, 16)]
                out_v[qi, pl.ds(0, 16)] = acc0
                out_v[qi, pl.ds(16, 16)] = acc1
                return c2

            lax.fori_loop(0, CH, qstep, 0)
            pltpu.sync_copy(out_v, out_hbm.at[w, k])
            return carry

        lax.fori_loop(0, NCHUNK, chunk, 0)

    return body(vt_flat, idxr, wtr)


CQ = 1200


def _out_proj_body(s_ref, wt_ref, b_ref, o_ref):
    acc = jnp.dot(s_ref[...], wt_ref[...], preferred_element_type=jnp.float32)
    o_ref[...] = acc + b_ref[...][None, :]


def _out_proj(sampled2d, w_out_t, b_out):
    return pl.pallas_call(
        _out_proj_body,
        grid=(BS * NQ // CQ,),
        in_specs=[
            pl.BlockSpec((CQ, EMBED), lambda i: (i, 0)),
            pl.BlockSpec((EMBED, EMBED), lambda i: (0, 0)),
            pl.BlockSpec((EMBED,), lambda i: (0,)),
        ],
        out_specs=pl.BlockSpec((CQ, EMBED), lambda i: (i, 0)),
        out_shape=jax.ShapeDtypeStruct((BS * NQ, EMBED), jnp.float32),
    )(sampled2d, w_out_t, b_out)


def kernel(query, value, reference_points, spatial_shapes,
           W_off, b_off, W_attn, b_attn, W_val, b_val, W_out, b_out):
    f32 = jnp.float32
    j32 = jnp.arange(NH * NP)
    wx = W_off[j32 * 2].astype(f32)
    bx = b_off[j32 * 2].astype(f32)
    wy = W_off[j32 * 2 + 1].astype(f32)
    by = b_off[j32 * 2 + 1].astype(f32)
    wa = W_attn.astype(f32)
    ba = b_attn.astype(f32)
    head32a = j32 // NP
    g = (head32a[:, None] == head32a[None, :]).astype(f32)

    vt = _patch_table(value, W_val.T.astype(f32), b_val.astype(f32))
    vt_flat = vt.reshape(BS * NH * HW, PATCH)

    ref_pts = reference_points[:, :, 0, :]
    idx, wt = _query_side(query, ref_pts, wx, bx, wy, by, wa, ba, g)

    idxr = (idx.reshape(BS, NQ, NH, NP).transpose(0, 2, 1, 3)
            .reshape(NW, NCHUNK, GJ, GB))
    wtr = (wt.reshape(BS, NQ, 4, NH, NP).transpose(0, 3, 1, 2, 4)
           .reshape(NW, NCHUNK, CH, NP * 4))

    sc = _sc_gather_reduce(vt_flat, idxr, wtr)
    sampled = (sc.reshape(BS, NH, NQ, HD).transpose(0, 2, 1, 3)
               .reshape(BS, NQ, EMBED))

    out2d = _out_proj(sampled.reshape(BS * NQ, EMBED),
                      W_out.T.astype(f32), b_out.astype(f32))
    return out2d.reshape(BS, NQ, EMBED)


# speedup vs baseline: 2.4135x; 1.5126x over previous
import functools

import jax
import jax.numpy as jnp
from jax import lax
from jax.experimental import pallas as pl
from jax.experimental.pallas import tpu as pltpu
from jax.experimental.pallas import tpu_sc as plsc

EMBED = 256
NH = 8
NP = 4
H_ = 100
W_ = 100
HW = H_ * W_
BS = 4
NQ = 900
HD = EMBED // NH
NW = 32
CH = 120
NFULL = NQ // CH
CTAIL = NQ - NFULL * CH
NTERM = NP * 4

POS_TILE = 1000


def _value_proj_body(v_ref, wt_ref, b_ref, out_ref):
    acc = jnp.dot(v_ref[0], wt_ref[...], preferred_element_type=jnp.float32)
    out_ref[0] = acc + b_ref[...][None, :]


def _value_proj(value, w_val_t, b_val):
    return pl.pallas_call(
        _value_proj_body,
        grid=(BS, HW // POS_TILE),
        in_specs=[
            pl.BlockSpec((1, POS_TILE, EMBED), lambda b, t: (b, t, 0)),
            pl.BlockSpec((EMBED, EMBED), lambda b, t: (0, 0)),
            pl.BlockSpec((EMBED,), lambda b, t: (0,)),
        ],
        out_specs=pl.BlockSpec((1, POS_TILE, EMBED), lambda b, t: (b, t, 0)),
        out_shape=jax.ShapeDtypeStruct((BS, HW, EMBED), jnp.float32),
    )(value, w_val_t, b_val)


def _query_side_body(q_ref, r_ref, wx_ref, bx_ref, wy_ref, by_ref,
                     wa_ref, ba_ref, g_ref, idx_ref, wt_ref):
    b = pl.program_id(0)
    q = q_ref[0]
    refx = r_ref[0, :, 0:1]
    refy = r_ref[0, :, 1:2]

    px = jnp.dot(q, wx_ref[...].T, preferred_element_type=jnp.float32)
    px = px + bx_ref[...][None, :] + (refx * W_ - 0.5)
    py = jnp.dot(q, wy_ref[...].T, preferred_element_type=jnp.float32)
    py = py + by_ref[...][None, :] + (refy * H_ - 0.5)

    x0 = jnp.clip(jnp.floor(px), 0.0, W_ - 2.0)
    y0 = jnp.clip(jnp.floor(py), 0.0, H_ - 2.0)

    head32 = lax.broadcasted_iota(jnp.int32, (NQ, NH * NP), 1) // NP
    x0i = x0.astype(jnp.int32)
    y0i = y0.astype(jnp.int32)

    logits = jnp.dot(q, wa_ref[...].T, preferred_element_type=jnp.float32)
    logits = logits + ba_ref[...][None, :]
    m = jnp.max(logits, axis=1, keepdims=True)
    e = jnp.exp(logits - m)
    s = jnp.dot(e, g_ref[...], preferred_element_type=jnp.float32)
    aw = e / s

    wparts = []
    iparts = []
    for sub in range(4):
        sx = sub % 2
        sy = sub // 2
        tx = jnp.maximum(0.0, 1.0 - jnp.abs(px - (x0 + float(sx))))
        ty = jnp.maximum(0.0, 1.0 - jnp.abs(py - (y0 + float(sy))))
        wparts.append(aw * tx * ty)
        pos = (y0i + sy) * W_ + (x0i + sx)
        iparts.append((b * HW + pos) * NH + head32)
    idx_ref[0] = jnp.concatenate(iparts, axis=1)
    wt_ref[0] = jnp.concatenate(wparts, axis=1)


def _query_side(query, ref_pts, wx, bx, wy, by, wa, ba, g):
    n128 = NH * NP * 4
    return pl.pallas_call(
        _query_side_body,
        grid=(BS,),
        in_specs=[
            pl.BlockSpec((1, NQ, EMBED), lambda b: (b, 0, 0)),
            pl.BlockSpec((1, NQ, 2), lambda b: (b, 0, 0)),
            pl.BlockSpec((NH * NP, EMBED), lambda b: (0, 0)),
            pl.BlockSpec((NH * NP,), lambda b: (0,)),
            pl.BlockSpec((NH * NP, EMBED), lambda b: (0, 0)),
            pl.BlockSpec((NH * NP,), lambda b: (0,)),
            pl.BlockSpec((NH * NP, EMBED), lambda b: (0, 0)),
            pl.BlockSpec((NH * NP,), lambda b: (0,)),
            pl.BlockSpec((NH * NP, NH * NP), lambda b: (0, 0)),
        ],
        out_specs=[
            pl.BlockSpec((1, NQ, n128), lambda b: (b, 0, 0)),
            pl.BlockSpec((1, NQ, n128), lambda b: (b, 0, 0)),
        ],
        out_shape=[
            jax.ShapeDtypeStruct((BS, NQ, n128), jnp.int32),
            jax.ShapeDtypeStruct((BS, NQ, n128), jnp.float32),
        ],
    )(query, ref_pts, wx, bx, wy, by, wa, ba, g)


def _sc_gather_reduce(vt_flat, idxr, wtr):
    mesh = plsc.VectorSubcoreMesh(core_axis_name="c", subcore_axis_name="s")

    @functools.partial(
        pl.kernel,
        mesh=mesh,
        compiler_params=pltpu.CompilerParams(use_tc_tiling_on_sc=False),
        out_type=jax.ShapeDtypeStruct((BS, NQ, EMBED), jnp.float32),
        scratch_types=[
            pltpu.VMEM((NP, NP, CH), jnp.int32),
            pltpu.VMEM((NTERM * CH, HD), jnp.float32),
            pltpu.VMEM((CH, NTERM), jnp.float32),
            pltpu.VMEM((CH, HD), jnp.float32),
            pltpu.SemaphoreType.DMA,
        ],
    )
    def body(vt_hbm, idx_hbm, wt_hbm, out_hbm, idx_v, rows_v, wt_v, out_v, sem):
        w = lax.axis_index("s") * 2 + lax.axis_index("c")
        b = w // NH
        h = w % NH

        def do_chunk(q0, gl, ql):
            pltpu.sync_copy(
                idx_hbm.at[b, :, h, :, pl.ds(q0, gl)],
                idx_v.at[:, :, pl.ds(0, gl)])
            pltpu.sync_copy(wt_hbm.at[w, pl.ds(q0, gl), :],
                            wt_v.at[pl.ds(0, gl)])
            handles = []
            for si in range(4):
                for pi in range(NP):
                    t = si * NP + pi
                    handles.append(pltpu.async_copy(
                        vt_hbm.at[idx_v.at[si, pi, pl.ds(0, gl)]],
                        rows_v.at[pl.ds(t * gl, gl)], sem))
            for hnd in handles:
                hnd.wait()

            def qstep(qi, c2):
                wrow = wt_v[qi, :]
                acc0 = jnp.zeros((16,), jnp.float32)
                acc1 = jnp.zeros((16,), jnp.float32)
                for t in range(NTERM):
                    wsc = wrow[t]
                    acc0 = acc0 + wsc * rows_v[t * gl + qi, pl.ds(0, 16)]
                    acc1 = acc1 + wsc * rows_v[t * gl + qi, pl.ds(16, 16)]
                out_v[qi, pl.ds(0, 16)] = acc0
                out_v[qi, pl.ds(16, 16)] = acc1
                return c2

            lax.fori_loop(0, ql, qstep, 0)
            pltpu.sync_copy(
                out_v.at[pl.ds(0, ql)],
                out_hbm.at[b, pl.ds(q0, ql), pl.ds(h * HD, HD)])

        def chunk(k, carry):
            do_chunk(k * CH, CH, CH)
            return carry

        lax.fori_loop(0, NFULL, chunk, 0)
        do_chunk(NFULL * CH, 64, CTAIL)

    return body(vt_flat, idxr, wtr)


CQ = 1200


def _out_proj_body(s_ref, wt_ref, b_ref, o_ref):
    acc = jnp.dot(s_ref[...], wt_ref[...], preferred_element_type=jnp.float32)
    o_ref[...] = acc + b_ref[...][None, :]


def _out_proj(sampled2d, w_out_t, b_out):
    return pl.pallas_call(
        _out_proj_body,
        grid=(BS * NQ // CQ,),
        in_specs=[
            pl.BlockSpec((CQ, EMBED), lambda i: (i, 0)),
            pl.BlockSpec((EMBED, EMBED), lambda i: (0, 0)),
            pl.BlockSpec((EMBED,), lambda i: (0,)),
        ],
        out_specs=pl.BlockSpec((CQ, EMBED), lambda i: (i, 0)),
        out_shape=jax.ShapeDtypeStruct((BS * NQ, EMBED), jnp.float32),
    )(sampled2d, w_out_t, b_out)


def kernel(query, value, reference_points, spatial_shapes,
           W_off, b_off, W_attn, b_attn, W_val, b_val, W_out, b_out):
    f32 = jnp.float32
    j32 = jnp.arange(NH * NP)
    wx = W_off[j32 * 2].astype(f32)
    bx = b_off[j32 * 2].astype(f32)
    wy = W_off[j32 * 2 + 1].astype(f32)
    by = b_off[j32 * 2 + 1].astype(f32)
    wa = W_attn.astype(f32)
    ba = b_attn.astype(f32)
    head32a = j32 // NP
    g = (head32a[:, None] == head32a[None, :]).astype(f32)

    vproj = _value_proj(value, W_val.T.astype(f32), b_val.astype(f32))
    vt_flat = vproj.reshape(BS * HW * NH, HD)

    ref_pts = reference_points[:, :, 0, :]
    idx, wt = _query_side(query, ref_pts, wx, bx, wy, by, wa, ba, g)

    idxr = idx.transpose(0, 2, 1).reshape(BS, 4, NH, NP, NQ)
    idxr = jnp.pad(idxr, ((0, 0), (0, 0), (0, 0), (0, 0), (0, 4)))
    wtr = (wt.reshape(BS, NQ, 4, NH, NP).transpose(0, 3, 1, 2, 4)
           .reshape(NW, NQ, NTERM))
    wtr = jnp.pad(wtr, ((0, 0), (0, 4), (0, 0)))

    sampled = _sc_gather_reduce(vt_flat, idxr, wtr)

    out2d = _out_proj(sampled.reshape(BS * NQ, EMBED),
                      W_out.T.astype(f32), b_out.astype(f32))
    return out2d.reshape(BS, NQ, EMBED)


# speedup vs baseline: 2.4267x; 1.0054x over previous
import functools

import jax
import jax.numpy as jnp
from jax import lax
from jax.experimental import pallas as pl
from jax.experimental.pallas import tpu as pltpu
from jax.experimental.pallas import tpu_sc as plsc

EMBED = 256
NH = 8
NP = 4
H_ = 100
W_ = 100
HW = H_ * W_
BS = 4
NQ = 900
HD = EMBED // NH
NW = 32
CH = 120
NFULL = NQ // CH
CTAIL = NQ - NFULL * CH
NTERM = NP * 4

POS_TILE = 1000


def _value_proj_body(v_ref, wt_ref, b_ref, out_ref):
    acc = jnp.dot(v_ref[0], wt_ref[...], preferred_element_type=jnp.float32)
    out_ref[0] = acc + b_ref[...][None, :]


def _value_proj(value, w_val_t, b_val):
    return pl.pallas_call(
        _value_proj_body,
        grid=(BS, HW // POS_TILE),
        in_specs=[
            pl.BlockSpec((1, POS_TILE, EMBED), lambda b, t: (b, t, 0)),
            pl.BlockSpec((EMBED, EMBED), lambda b, t: (0, 0)),
            pl.BlockSpec((EMBED,), lambda b, t: (0,)),
        ],
        out_specs=pl.BlockSpec((1, POS_TILE, EMBED), lambda b, t: (b, t, 0)),
        out_shape=jax.ShapeDtypeStruct((BS, HW, EMBED), jnp.float32),
    )(value, w_val_t, b_val)


def _query_side_body(q_ref, r_ref, wx_ref, bx_ref, wy_ref, by_ref,
                     wa_ref, ba_ref, g_ref, idx_ref, wt_ref):
    b = pl.program_id(0)
    q = q_ref[0]
    refx = r_ref[0, :, 0:1]
    refy = r_ref[0, :, 1:2]

    px = jnp.dot(q, wx_ref[...].T, preferred_element_type=jnp.float32)
    px = px + bx_ref[...][None, :] + (refx * W_ - 0.5)
    py = jnp.dot(q, wy_ref[...].T, preferred_element_type=jnp.float32)
    py = py + by_ref[...][None, :] + (refy * H_ - 0.5)

    x0 = jnp.clip(jnp.floor(px), 0.0, W_ - 2.0)
    y0 = jnp.clip(jnp.floor(py), 0.0, H_ - 2.0)

    head32 = lax.broadcasted_iota(jnp.int32, (NQ, NH * NP), 1) // NP
    x0i = x0.astype(jnp.int32)
    y0i = y0.astype(jnp.int32)

    logits = jnp.dot(q, wa_ref[...].T, preferred_element_type=jnp.float32)
    logits = logits + ba_ref[...][None, :]
    m = jnp.max(logits, axis=1, keepdims=True)
    e = jnp.exp(logits - m)
    s = jnp.dot(e, g_ref[...], preferred_element_type=jnp.float32)
    aw = e / s

    wplanes = []
    iplanes = []
    for sub in range(4):
        sx = sub % 2
        sy = sub // 2
        tx = jnp.maximum(0.0, 1.0 - jnp.abs(px - (x0 + float(sx))))
        ty = jnp.maximum(0.0, 1.0 - jnp.abs(py - (y0 + float(sy))))
        wplanes.append(aw * tx * ty)
        pos = (y0i + sy) * W_ + (x0i + sx)
        iplanes.append((b * HW + pos) * NH + head32)
    wpieces = []
    ipieces = []
    for h in range(NH):
        for sub in range(4):
            sl = slice(h * NP, (h + 1) * NP)
            wpieces.append(wplanes[sub][:, sl])
            ipieces.append(iplanes[sub][:, sl])
    idx_ref[0] = jnp.concatenate(ipieces, axis=1)
    wt_ref[0] = jnp.concatenate(wpieces, axis=1)


def _query_side(query, ref_pts, wx, bx, wy, by, wa, ba, g):
    n128 = NH * NP * 4
    return pl.pallas_call(
        _query_side_body,
        grid=(BS,),
        in_specs=[
            pl.BlockSpec((1, NQ, EMBED), lambda b: (b, 0, 0)),
            pl.BlockSpec((1, NQ, 2), lambda b: (b, 0, 0)),
            pl.BlockSpec((NH * NP, EMBED), lambda b: (0, 0)),
            pl.BlockSpec((NH * NP,), lambda b: (0,)),
            pl.BlockSpec((NH * NP, EMBED), lambda b: (0, 0)),
            pl.BlockSpec((NH * NP,), lambda b: (0,)),
            pl.BlockSpec((NH * NP, EMBED), lambda b: (0, 0)),
            pl.BlockSpec((NH * NP,), lambda b: (0,)),
            pl.BlockSpec((NH * NP, NH * NP), lambda b: (0, 0)),
        ],
        out_specs=[
            pl.BlockSpec((1, NQ, n128), lambda b: (b, 0, 0)),
            pl.BlockSpec((1, NQ, n128), lambda b: (b, 0, 0)),
        ],
        out_shape=[
            jax.ShapeDtypeStruct((BS, NQ, n128), jnp.int32),
            jax.ShapeDtypeStruct((BS, NQ, n128), jnp.float32),
        ],
    )(query, ref_pts, wx, bx, wy, by, wa, ba, g)


def _sc_gather_reduce(vt_flat, idxr, wtr):
    mesh = plsc.VectorSubcoreMesh(core_axis_name="c", subcore_axis_name="s")

    @functools.partial(
        pl.kernel,
        mesh=mesh,
        compiler_params=pltpu.CompilerParams(use_tc_tiling_on_sc=False),
        out_type=jax.ShapeDtypeStruct((BS, NQ, EMBED), jnp.float32),
        scratch_types=[
            pltpu.VMEM((CH, NTERM), jnp.int32),
            pltpu.VMEM((CH * NTERM,), jnp.int32),
            pltpu.VMEM((CH * NTERM, HD), jnp.float32),
            pltpu.VMEM((CH, NTERM), jnp.float32),
            pltpu.VMEM((CH, HD), jnp.float32),
            pltpu.SemaphoreType.DMA,
        ],
    )
    def body(vt_hbm, idx_hbm, wt_hbm, out_hbm,
             idx_v, flat_v, rows_v, wt_v, out_v, sem):
        w = lax.axis_index("s") * 2 + lax.axis_index("c")
        b = w // NH
        h = w % NH

        def do_chunk(q0, gl, ql):
            pltpu.sync_copy(
                idx_hbm.at[b, pl.ds(q0, gl), pl.ds(h * NTERM, NTERM)],
                idx_v.at[pl.ds(0, gl)])
            pltpu.sync_copy(
                wt_hbm.at[b, pl.ds(q0, gl), pl.ds(h * NTERM, NTERM)],
                wt_v.at[pl.ds(0, gl)])

            def repack(qi, c2):
                flat_v[pl.ds(qi * NTERM, NTERM)] = idx_v[qi, :]
                return c2

            lax.fori_loop(0, gl, repack, 0)
            handles = []
            for j in range(gl * NTERM // 128):
                handles.append(pltpu.async_copy(
                    vt_hbm.at[flat_v.at[pl.ds(j * 128, 128)]],
                    rows_v.at[pl.ds(j * 128, 128)], sem))
            for hnd in handles:
                hnd.wait()

            def qstep(qi, c2):
                wrow = wt_v[qi, :]
                acc0 = jnp.zeros((16,), jnp.float32)
                acc1 = jnp.zeros((16,), jnp.float32)
                for t in range(NTERM):
                    wsc = wrow[t]
                    acc0 = acc0 + wsc * rows_v[qi * NTERM + t, pl.ds(0, 16)]
                    acc1 = acc1 + wsc * rows_v[qi * NTERM + t, pl.ds(16, 16)]
                out_v[qi, pl.ds(0, 16)] = acc0
                out_v[qi, pl.ds(16, 16)] = acc1
                return c2

            lax.fori_loop(0, ql, qstep, 0)
            pltpu.sync_copy(
                out_v.at[pl.ds(0, ql)],
                out_hbm.at[b, pl.ds(q0, ql), pl.ds(h * HD, HD)])

        def chunk(k, carry):
            do_chunk(k * CH, CH, CH)
            return carry

        lax.fori_loop(0, NFULL, chunk, 0)
        do_chunk(NFULL * CH, 64, CTAIL)

    return body(vt_flat, idxr, wtr)


CQ = 1200


def _out_proj_body(s_ref, wt_ref, b_ref, o_ref):
    acc = jnp.dot(s_ref[...], wt_ref[...], preferred_element_type=jnp.float32)
    o_ref[...] = acc + b_ref[...][None, :]


def _out_proj(sampled2d, w_out_t, b_out):
    return pl.pallas_call(
        _out_proj_body,
        grid=(BS * NQ // CQ,),
        in_specs=[
            pl.BlockSpec((CQ, EMBED), lambda i: (i, 0)),
            pl.BlockSpec((EMBED, EMBED), lambda i: (0, 0)),
            pl.BlockSpec((EMBED,), lambda i: (0,)),
        ],
        out_specs=pl.BlockSpec((CQ, EMBED), lambda i: (i, 0)),
        out_shape=jax.ShapeDtypeStruct((BS * NQ, EMBED), jnp.float32),
    )(sampled2d, w_out_t, b_out)


def kernel(query, value, reference_points, spatial_shapes,
           W_off, b_off, W_attn, b_attn, W_val, b_val, W_out, b_out):
    f32 = jnp.float32
    j32 = jnp.arange(NH * NP)
    wx = W_off[j32 * 2].astype(f32)
    bx = b_off[j32 * 2].astype(f32)
    wy = W_off[j32 * 2 + 1].astype(f32)
    by = b_off[j32 * 2 + 1].astype(f32)
    wa = W_attn.astype(f32)
    ba = b_attn.astype(f32)
    head32a = j32 // NP
    g = (head32a[:, None] == head32a[None, :]).astype(f32)

    vproj = _value_proj(value, W_val.T.astype(f32), b_val.astype(f32))
    vt_flat = vproj.reshape(BS * HW * NH, HD)

    ref_pts = reference_points[:, :, 0, :]
    idx, wt = _query_side(query, ref_pts, wx, bx, wy, by, wa, ba, g)

    idxr = jnp.pad(idx, ((0, 0), (0, 4), (0, 0)))
    wtr = jnp.pad(wt, ((0, 0), (0, 4), (0, 0)))

    sampled = _sc_gather_reduce(vt_flat, idxr, wtr)

    out2d = _out_proj(sampled.reshape(BS * NQ, EMBED),
                      W_out.T.astype(f32), b_out.astype(f32))
    return out2d.reshape(BS, NQ, EMBED)


# speedup vs baseline: 2.8741x; 1.1844x over previous
import functools

import jax
import jax.numpy as jnp
from jax import lax
from jax.experimental import pallas as pl
from jax.experimental.pallas import tpu as pltpu
from jax.experimental.pallas import tpu_sc as plsc

EMBED = 256
NH = 8
NP = 4
H_ = 100
W_ = 100
HW = H_ * W_
BS = 4
NQ = 900
HD = EMBED // NH
NW = 32
CH = 120
NFULL = NQ // CH
CTAIL = NQ - NFULL * CH
NTERM = NP * 4

POS_TILE = 1000


def _value_proj_body(v_ref, wt_ref, b_ref, out_ref):
    acc = jnp.dot(v_ref[0], wt_ref[...], preferred_element_type=jnp.float32)
    acc = acc + b_ref[...][None, :]
    out_ref[0] = acc[:, 0:128]
    out_ref[1] = acc[:, 128:256]


def _value_proj(value, w_val_t, b_val):
    nt = HW // POS_TILE
    return pl.pallas_call(
        _value_proj_body,
        grid=(BS, nt),
        in_specs=[
            pl.BlockSpec((1, POS_TILE, EMBED), lambda b, t: (b, t, 0)),
            pl.BlockSpec((EMBED, EMBED), lambda b, t: (0, 0)),
            pl.BlockSpec((EMBED,), lambda b, t: (0,)),
        ],
        out_specs=pl.BlockSpec((2, POS_TILE, 128),
                               lambda b, t: (0, b * nt + t, 0)),
        out_shape=jax.ShapeDtypeStruct((2, BS * HW, 128), jnp.float32),
    )(value, w_val_t, b_val)


def _query_side_body(q_ref, r_ref, wx_ref, bx_ref, wy_ref, by_ref,
                     wa_ref, ba_ref, g_ref, p_ref, idx_ref, wt_ref):
    b = pl.program_id(0)
    q = q_ref[0]
    refx = r_ref[0, :, 0:1]
    refy = r_ref[0, :, 1:2]

    px = jnp.dot(q, wx_ref[...].T, preferred_element_type=jnp.float32)
    px = px + bx_ref[...][None, :] + (refx * W_ - 0.5)
    py = jnp.dot(q, wy_ref[...].T, preferred_element_type=jnp.float32)
    py = py + by_ref[...][None, :] + (refy * H_ - 0.5)

    x0 = jnp.clip(jnp.floor(px), 0.0, W_ - 2.0)
    y0 = jnp.clip(jnp.floor(py), 0.0, H_ - 2.0)

    head32 = lax.broadcasted_iota(jnp.int32, (NQ, NH * NP), 1) // NP
    x0i = x0.astype(jnp.int32)
    y0i = y0.astype(jnp.int32)

    logits = jnp.dot(q, wa_ref[...].T, preferred_element_type=jnp.float32)
    logits = logits + ba_ref[...][None, :]
    m = jnp.max(logits, axis=1, keepdims=True)
    e = jnp.exp(logits - m)
    s = jnp.dot(e, g_ref[...], preferred_element_type=jnp.float32)
    aw = e / s

    x128 = jnp.zeros((NQ, NH * NTERM), jnp.float32)
    y128 = jnp.zeros((NQ, NH * NTERM), jnp.float32)
    w128 = jnp.zeros((NQ, NH * NTERM), jnp.float32)
    for sub in range(4):
        sx = sub % 2
        sy = sub // 2
        tx = jnp.maximum(0.0, 1.0 - jnp.abs(px - (x0 + float(sx))))
        ty = jnp.maximum(0.0, 1.0 - jnp.abs(py - (y0 + float(sy))))
        perm = p_ref[sub]
        w128 = w128 + jnp.dot(aw * tx * ty, perm,
                              preferred_element_type=jnp.float32)
        x128 = x128 + jnp.dot(x0 + float(sx), perm,
                              preferred_element_type=jnp.float32)
        y128 = y128 + jnp.dot(y0 + float(sy), perm,
                              preferred_element_type=jnp.float32)
    wt_ref[0] = w128
    head128 = lax.broadcasted_iota(jnp.int32, (NQ, NH * NTERM), 1) // NTERM
    pos = y128.astype(jnp.int32) * W_ + x128.astype(jnp.int32)
    idx_ref[0] = ((head128 // 4) * (BS * HW * 4)
                  + (b * HW + pos) * 4 + head128 % 4)


def _query_side(query, ref_pts, wx, bx, wy, by, wa, ba, g, perms):
    n128 = NH * NP * 4
    return pl.pallas_call(
        _query_side_body,
        grid=(BS,),
        in_specs=[
            pl.BlockSpec((1, NQ, EMBED), lambda b: (b, 0, 0)),
            pl.BlockSpec((1, NQ, 2), lambda b: (b, 0, 0)),
            pl.BlockSpec((NH * NP, EMBED), lambda b: (0, 0)),
            pl.BlockSpec((NH * NP,), lambda b: (0,)),
            pl.BlockSpec((NH * NP, EMBED), lambda b: (0, 0)),
            pl.BlockSpec((NH * NP,), lambda b: (0,)),
            pl.BlockSpec((NH * NP, EMBED), lambda b: (0, 0)),
            pl.BlockSpec((NH * NP,), lambda b: (0,)),
            pl.BlockSpec((NH * NP, NH * NP), lambda b: (0, 0)),
            pl.BlockSpec((4, NH * NP, n128), lambda b: (0, 0, 0)),
        ],
        out_specs=[
            pl.BlockSpec((1, NQ, n128), lambda b: (b, 0, 0)),
            pl.BlockSpec((1, NQ, n128), lambda b: (b, 0, 0)),
        ],
        out_shape=[
            jax.ShapeDtypeStruct((BS, NQ, n128), jnp.int32),
            jax.ShapeDtypeStruct((BS, NQ, n128), jnp.float32),
        ],
    )(query, ref_pts, wx, bx, wy, by, wa, ba, g, perms)


def _sc_gather_reduce(vt_flat, idxr, wtr):
    mesh = plsc.VectorSubcoreMesh(core_axis_name="c", subcore_axis_name="s")

    @functools.partial(
        pl.kernel,
        mesh=mesh,
        compiler_params=pltpu.CompilerParams(use_tc_tiling_on_sc=False),
        out_type=jax.ShapeDtypeStruct((BS, NQ, EMBED), jnp.float32),
        scratch_types=[
            pltpu.VMEM((CH, NTERM), jnp.int32),
            pltpu.VMEM((CH * NTERM,), jnp.int32),
            pltpu.VMEM((CH * NTERM, HD), jnp.float32),
            pltpu.VMEM((CH, NTERM), jnp.float32),
            pltpu.VMEM((CH, HD), jnp.float32),
            pltpu.SemaphoreType.DMA,
        ],
    )
    def body(vt_hbm, idx_hbm, wt_hbm, out_hbm,
             idx_v, flat_v, rows_v, wt_v, out_v, sem):
        w = lax.axis_index("s") * 2 + lax.axis_index("c")
        b = w // NH
        h = w % NH

        def do_chunk(q0, gl, ql):
            pltpu.sync_copy(
                idx_hbm.at[b, pl.ds(q0, gl), pl.ds(h * NTERM, NTERM)],
                idx_v.at[pl.ds(0, gl)])
            pltpu.sync_copy(
                wt_hbm.at[b, pl.ds(q0, gl), pl.ds(h * NTERM, NTERM)],
                wt_v.at[pl.ds(0, gl)])

            def repack(qi, c2):
                flat_v[pl.ds(qi * NTERM, NTERM)] = idx_v[qi, :]
                return c2

            lax.fori_loop(0, gl, repack, 0)
            handles = []
            for j in range(gl * NTERM // 128):
                handles.append(pltpu.async_copy(
                    vt_hbm.at[flat_v.at[pl.ds(j * 128, 128)]],
                    rows_v.at[pl.ds(j * 128, 128)], sem))
            for hnd in handles:
                hnd.wait()

            def qstep(qi, c2):
                wrow = wt_v[qi, :]
                acc0 = jnp.zeros((16,), jnp.float32)
                acc1 = jnp.zeros((16,), jnp.float32)
                for t in range(NTERM):
                    wsc = wrow[t]
                    acc0 = acc0 + wsc * rows_v[qi * NTERM + t, pl.ds(0, 16)]
                    acc1 = acc1 + wsc * rows_v[qi * NTERM + t, pl.ds(16, 16)]
                out_v[qi, pl.ds(0, 16)] = acc0
                out_v[qi, pl.ds(16, 16)] = acc1
                return c2

            lax.fori_loop(0, ql, qstep, 0)
            pltpu.sync_copy(
                out_v.at[pl.ds(0, ql)],
                out_hbm.at[b, pl.ds(q0, ql), pl.ds(h * HD, HD)])

        def chunk(k, carry):
            do_chunk(k * CH, CH, CH)
            return carry

        lax.fori_loop(0, NFULL, chunk, 0)
        do_chunk(NFULL * CH, 64, CTAIL)

    return body(vt_flat, idxr, wtr)


CQ = 1200


def _out_proj_body(s_ref, wt_ref, b_ref, o_ref):
    acc = jnp.dot(s_ref[...], wt_ref[...], preferred_element_type=jnp.float32)
    o_ref[...] = acc + b_ref[...][None, :]


def _out_proj(sampled2d, w_out_t, b_out):
    return pl.pallas_call(
        _out_proj_body,
        grid=(BS * NQ // CQ,),
        in_specs=[
            pl.BlockSpec((CQ, EMBED), lambda i: (i, 0)),
            pl.BlockSpec((EMBED, EMBED), lambda i: (0, 0)),
            pl.BlockSpec((EMBED,), lambda i: (0,)),
        ],
        out_specs=pl.BlockSpec((CQ, EMBED), lambda i: (i, 0)),
        out_shape=jax.ShapeDtypeStruct((BS * NQ, EMBED), jnp.float32),
    )(sampled2d, w_out_t, b_out)


def kernel(query, value, reference_points, spatial_shapes,
           W_off, b_off, W_attn, b_attn, W_val, b_val, W_out, b_out):
    f32 = jnp.float32
    j32 = jnp.arange(NH * NP)
    wx = W_off[j32 * 2].astype(f32)
    bx = b_off[j32 * 2].astype(f32)
    wy = W_off[j32 * 2 + 1].astype(f32)
    by = b_off[j32 * 2 + 1].astype(f32)
    wa = W_attn.astype(f32)
    ba = b_attn.astype(f32)
    head32a = j32 // NP
    g = (head32a[:, None] == head32a[None, :]).astype(f32)
    s4 = jnp.arange(4)[:, None, None]
    j128b = jnp.arange(NH * NTERM)[None, None, :]
    c32 = j32[None, :, None]
    perms = (j128b == (c32 // NP) * NTERM + s4 * NP + c32 % NP).astype(f32)

    vproj = _value_proj(value, W_val.T.astype(f32), b_val.astype(f32))
    vt_flat = vproj.reshape(BS * HW * NH, HD)

    ref_pts = reference_points[:, :, 0, :]
    idx, wt = _query_side(query, ref_pts, wx, bx, wy, by, wa, ba, g, perms)

    idxr = jnp.pad(idx, ((0, 0), (0, 4), (0, 0)))
    wtr = jnp.pad(wt, ((0, 0), (0, 4), (0, 0)))

    sampled = _sc_gather_reduce(vt_flat, idxr, wtr)

    out2d = _out_proj(sampled.reshape(BS * NQ, EMBED),
                      W_out.T.astype(f32), b_out.astype(f32))
    return out2d.reshape(BS, NQ, EMBED)


# speedup vs baseline: 3.1620x; 1.1002x over previous
import functools

import jax
import jax.numpy as jnp
from jax import lax
from jax.experimental import pallas as pl
from jax.experimental.pallas import tpu as pltpu
from jax.experimental.pallas import tpu_sc as plsc

EMBED = 256
NH = 8
NP = 4
H_ = 100
W_ = 100
HW = H_ * W_
BS = 4
NQ = 900
HD = EMBED // NH
NW = 32
CH = 96
NTERM = NP * 4
SC_CHUNKS = [(k * CH, CH, CH) for k in range(NQ // CH)] + [(864, 40, 36)]

POS_TILE = 1000


def _value_proj_body(v_ref, wt_ref, b_ref, out_ref):
    acc = jnp.dot(v_ref[0], wt_ref[...], preferred_element_type=jnp.float32)
    acc = acc + b_ref[...][None, :]
    out_ref[0] = acc[:, 0:128]
    out_ref[1] = acc[:, 128:256]


def _value_proj(value, w_val_t, b_val):
    nt = HW // POS_TILE
    return pl.pallas_call(
        _value_proj_body,
        grid=(BS, nt),
        in_specs=[
            pl.BlockSpec((1, POS_TILE, EMBED), lambda b, t: (b, t, 0)),
            pl.BlockSpec((EMBED, EMBED), lambda b, t: (0, 0)),
            pl.BlockSpec((EMBED,), lambda b, t: (0,)),
        ],
        out_specs=pl.BlockSpec((2, POS_TILE, 128),
                               lambda b, t: (0, b * nt + t, 0)),
        out_shape=jax.ShapeDtypeStruct((2, BS * HW, 128), jnp.float32),
    )(value, w_val_t, b_val)


def _query_side_body(q_ref, r_ref, wx_ref, bx_ref, wy_ref, by_ref,
                     wa_ref, ba_ref, g_ref, p_ref, idx_ref, wt_ref):
    b = pl.program_id(0)
    q = q_ref[0]
    refx = r_ref[0, :, 0:1]
    refy = r_ref[0, :, 1:2]

    px = jnp.dot(q, wx_ref[...].T, preferred_element_type=jnp.float32)
    px = px + bx_ref[...][None, :] + (refx * W_ - 0.5)
    py = jnp.dot(q, wy_ref[...].T, preferred_element_type=jnp.float32)
    py = py + by_ref[...][None, :] + (refy * H_ - 0.5)

    x0 = jnp.clip(jnp.floor(px), 0.0, W_ - 2.0)
    y0 = jnp.clip(jnp.floor(py), 0.0, H_ - 2.0)

    head32 = lax.broadcasted_iota(jnp.int32, (NQ, NH * NP), 1) // NP
    x0i = x0.astype(jnp.int32)
    y0i = y0.astype(jnp.int32)

    logits = jnp.dot(q, wa_ref[...].T, preferred_element_type=jnp.float32)
    logits = logits + ba_ref[...][None, :]
    m = jnp.max(logits, axis=1, keepdims=True)
    e = jnp.exp(logits - m)
    s = jnp.dot(e, g_ref[...], preferred_element_type=jnp.float32)
    aw = e / s

    x128 = jnp.zeros((NQ, NH * NTERM), jnp.float32)
    y128 = jnp.zeros((NQ, NH * NTERM), jnp.float32)
    w128 = jnp.zeros((NQ, NH * NTERM), jnp.float32)
    for sub in range(4):
        sx = sub % 2
        sy = sub // 2
        tx = jnp.maximum(0.0, 1.0 - jnp.abs(px - (x0 + float(sx))))
        ty = jnp.maximum(0.0, 1.0 - jnp.abs(py - (y0 + float(sy))))
        perm = p_ref[sub]
        w128 = w128 + jnp.dot(aw * tx * ty, perm,
                              preferred_element_type=jnp.float32)
        x128 = x128 + jnp.dot(x0 + float(sx), perm,
                              preferred_element_type=jnp.float32)
        y128 = y128 + jnp.dot(y0 + float(sy), perm,
                              preferred_element_type=jnp.float32)
    wt_ref[0] = w128
    head128 = lax.broadcasted_iota(jnp.int32, (NQ, NH * NTERM), 1) // NTERM
    pos = y128.astype(jnp.int32) * W_ + x128.astype(jnp.int32)
    idx_ref[0] = ((head128 // 4) * (BS * HW * 4)
                  + (b * HW + pos) * 4 + head128 % 4)


def _query_side(query, ref_pts, wx, bx, wy, by, wa, ba, g, perms):
    n128 = NH * NP * 4
    return pl.pallas_call(
        _query_side_body,
        grid=(BS,),
        in_specs=[
            pl.BlockSpec((1, NQ, EMBED), lambda b: (b, 0, 0)),
            pl.BlockSpec((1, NQ, 2), lambda b: (b, 0, 0)),
            pl.BlockSpec((NH * NP, EMBED), lambda b: (0, 0)),
            pl.BlockSpec((NH * NP,), lambda b: (0,)),
            pl.BlockSpec((NH * NP, EMBED), lambda b: (0, 0)),
            pl.BlockSpec((NH * NP,), lambda b: (0,)),
            pl.BlockSpec((NH * NP, EMBED), lambda b: (0, 0)),
            pl.BlockSpec((NH * NP,), lambda b: (0,)),
            pl.BlockSpec((NH * NP, NH * NP), lambda b: (0, 0)),
            pl.BlockSpec((4, NH * NP, n128), lambda b: (0, 0, 0)),
        ],
        out_specs=[
            pl.BlockSpec((1, NQ, n128), lambda b: (b, 0, 0)),
            pl.BlockSpec((1, NQ, n128), lambda b: (b, 0, 0)),
        ],
        out_shape=[
            jax.ShapeDtypeStruct((BS, NQ, n128), jnp.int32),
            jax.ShapeDtypeStruct((BS, NQ, n128), jnp.float32),
        ],
    )(query, ref_pts, wx, bx, wy, by, wa, ba, g, perms)


def _sc_gather_reduce(vt_flat, idxr, wtr):
    mesh = plsc.VectorSubcoreMesh(core_axis_name="c", subcore_axis_name="s")

    @functools.partial(
        pl.kernel,
        mesh=mesh,
        compiler_params=pltpu.CompilerParams(use_tc_tiling_on_sc=False),
        out_type=jax.ShapeDtypeStruct((BS, NQ, EMBED), jnp.float32),
        scratch_types=[
            pltpu.VMEM((2, CH, NTERM), jnp.int32),
            pltpu.VMEM((2, CH * NTERM), jnp.int32),
            pltpu.VMEM((2, CH * NTERM, HD), jnp.float32),
            pltpu.VMEM((2, CH, NTERM), jnp.float32),
            pltpu.VMEM((CH, HD), jnp.float32),
            pltpu.SemaphoreType.DMA,
            pltpu.SemaphoreType.DMA,
        ],
    )
    def body(vt_hbm, idx_hbm, wt_hbm, out_hbm,
             idx_v, flat_v, rows_v, wt_v, out_v, sem0, sem1):
        w = lax.axis_index("s") * 2 + lax.axis_index("c")
        b = w // NH
        h = w % NH
        sems = [sem0, sem1]

        def fetch(q0, gl, ib):
            pltpu.sync_copy(
                idx_hbm.at[b, pl.ds(q0, gl), pl.ds(h * NTERM, NTERM)],
                idx_v.at[ib, pl.ds(0, gl)])
            pltpu.sync_copy(
                wt_hbm.at[b, pl.ds(q0, gl), pl.ds(h * NTERM, NTERM)],
                wt_v.at[ib, pl.ds(0, gl)])

            def repack(qi, c2):
                flat_v[ib, pl.ds(qi * NTERM, NTERM)] = idx_v[ib, qi, :]
                return c2

            lax.fori_loop(0, gl, repack, 0)
            handles = []
            for j in range(gl * NTERM // 128):
                handles.append(pltpu.async_copy(
                    vt_hbm.at[flat_v.at[ib, pl.ds(j * 128, 128)]],
                    rows_v.at[ib, pl.ds(j * 128, 128)], sems[ib]))
            return handles

        def compute(q0, ql, ib):
            def qstep(qi, c2):
                wrow = wt_v[ib, qi, :]
                acc0 = jnp.zeros((16,), jnp.float32)
                acc1 = jnp.zeros((16,), jnp.float32)
                for t in range(NTERM):
                    wsc = wrow[t]
                    acc0 = acc0 + wsc * rows_v[ib, qi * NTERM + t,
                                               pl.ds(0, 16)]
                    acc1 = acc1 + wsc * rows_v[ib, qi * NTERM + t,
                                               pl.ds(16, 16)]
                out_v[qi, pl.ds(0, 16)] = acc0
                out_v[qi, pl.ds(16, 16)] = acc1
                return c2

            lax.fori_loop(0, ql, qstep, 0)
            pltpu.sync_copy(
                out_v.at[pl.ds(0, ql)],
                out_hbm.at[b, pl.ds(q0, ql), pl.ds(h * HD, HD)])

        handles = fetch(SC_CHUNKS[0][0], SC_CHUNKS[0][1], 0)
        for i, (q0, gl, ql) in enumerate(SC_CHUNKS):
            for hnd in handles:
                hnd.wait()
            if i + 1 < len(SC_CHUNKS):
                nq0, ngl, _ = SC_CHUNKS[i + 1]
                handles = fetch(nq0, ngl, (i + 1) % 2)
            compute(q0, ql, i % 2)

    return body(vt_flat, idxr, wtr)


CQ = 1200


def _out_proj_body(s_ref, wt_ref, b_ref, o_ref):
    acc = jnp.dot(s_ref[...], wt_ref[...], preferred_element_type=jnp.float32)
    o_ref[...] = acc + b_ref[...][None, :]


def _out_proj(sampled2d, w_out_t, b_out):
    return pl.pallas_call(
        _out_proj_body,
        grid=(BS * NQ // CQ,),
        in_specs=[
            pl.BlockSpec((CQ, EMBED), lambda i: (i, 0)),
            pl.BlockSpec((EMBED, EMBED), lambda i: (0, 0)),
            pl.BlockSpec((EMBED,), lambda i: (0,)),
        ],
        out_specs=pl.BlockSpec((CQ, EMBED), lambda i: (i, 0)),
        out_shape=jax.ShapeDtypeStruct((BS * NQ, EMBED), jnp.float32),
    )(sampled2d, w_out_t, b_out)


def kernel(query, value, reference_points, spatial_shapes,
           W_off, b_off, W_attn, b_attn, W_val, b_val, W_out, b_out):
    f32 = jnp.float32
    j32 = jnp.arange(NH * NP)
    wx = W_off[j32 * 2].astype(f32)
    bx = b_off[j32 * 2].astype(f32)
    wy = W_off[j32 * 2 + 1].astype(f32)
    by = b_off[j32 * 2 + 1].astype(f32)
    wa = W_attn.astype(f32)
    ba = b_attn.astype(f32)
    head32a = j32 // NP
    g = (head32a[:, None] == head32a[None, :]).astype(f32)
    s4 = jnp.arange(4)[:, None, None]
    j128b = jnp.arange(NH * NTERM)[None, None, :]
    c32 = j32[None, :, None]
    perms = (j128b == (c32 // NP) * NTERM + s4 * NP + c32 % NP).astype(f32)

    vproj = _value_proj(value, W_val.T.astype(f32), b_val.astype(f32))
    vt_flat = vproj.reshape(BS * HW * NH, HD)

    ref_pts = reference_points[:, :, 0, :]
    idx, wt = _query_side(query, ref_pts, wx, bx, wy, by, wa, ba, g, perms)

    idxr = jnp.pad(idx, ((0, 0), (0, 4), (0, 0)))
    wtr = jnp.pad(wt, ((0, 0), (0, 4), (0, 0)))

    sampled = _sc_gather_reduce(vt_flat, idxr, wtr)

    out2d = _out_proj(sampled.reshape(BS * NQ, EMBED),
                      W_out.T.astype(f32), b_out.astype(f32))
    return out2d.reshape(BS, NQ, EMBED)


# speedup vs baseline: 3.2680x; 1.0335x over previous
import functools

import jax
import jax.numpy as jnp
from jax import lax
from jax.experimental import pallas as pl
from jax.experimental.pallas import tpu as pltpu
from jax.experimental.pallas import tpu_sc as plsc

EMBED = 256
NH = 8
NP = 4
H_ = 100
W_ = 100
HW = H_ * W_
BS = 4
NQ = 900
HD = EMBED // NH
NW = 32
CH = 96
NTERM = NP * 4
SC_CHUNKS = [(k * CH, CH, CH) for k in range(NQ // CH)] + [(864, 40, 36)]

POS_TILE = 1000


def _value_proj_body(v_ref, wt_ref, b_ref, out_ref):
    acc = jnp.dot(v_ref[0], wt_ref[...], preferred_element_type=jnp.float32)
    acc = acc + b_ref[...][None, :]
    out_ref[0] = acc[:, 0:128]
    out_ref[1] = acc[:, 128:256]


def _value_proj(value, w_val_t, b_val):
    nt = HW // POS_TILE
    return pl.pallas_call(
        _value_proj_body,
        grid=(BS, nt),
        in_specs=[
            pl.BlockSpec((1, POS_TILE, EMBED), lambda b, t: (b, t, 0)),
            pl.BlockSpec((EMBED, EMBED), lambda b, t: (0, 0)),
            pl.BlockSpec((EMBED,), lambda b, t: (0,)),
        ],
        out_specs=pl.BlockSpec((2, POS_TILE, 128),
                               lambda b, t: (0, b * nt + t, 0)),
        out_shape=jax.ShapeDtypeStruct((2, BS * HW, 128), jnp.float32),
    )(value, w_val_t, b_val)


def _query_side_body(q_ref, r_ref, wx_ref, bx_ref, wy_ref, by_ref,
                     wa_ref, ba_ref, g_ref, p_ref, idx_ref, wt_ref):
    b = pl.program_id(0)
    q = q_ref[0]
    refx = r_ref[0, :, 0:1]
    refy = r_ref[0, :, 1:2]

    px = jnp.dot(q, wx_ref[...].T, preferred_element_type=jnp.float32)
    px = px + bx_ref[...][None, :] + (refx * W_ - 0.5)
    py = jnp.dot(q, wy_ref[...].T, preferred_element_type=jnp.float32)
    py = py + by_ref[...][None, :] + (refy * H_ - 0.5)

    x0 = jnp.clip(jnp.floor(px), 0.0, W_ - 2.0)
    y0 = jnp.clip(jnp.floor(py), 0.0, H_ - 2.0)

    head32 = lax.broadcasted_iota(jnp.int32, (NQ, NH * NP), 1) // NP
    x0i = x0.astype(jnp.int32)
    y0i = y0.astype(jnp.int32)

    logits = jnp.dot(q, wa_ref[...].T, preferred_element_type=jnp.float32)
    logits = logits + ba_ref[...][None, :]
    m = jnp.max(logits, axis=1, keepdims=True)
    e = jnp.exp(logits - m)
    s = jnp.dot(e, g_ref[...], preferred_element_type=jnp.float32)
    aw = e / s

    x128 = jnp.zeros((NQ, NH * NTERM), jnp.float32)
    y128 = jnp.zeros((NQ, NH * NTERM), jnp.float32)
    w128 = jnp.zeros((NQ, NH * NTERM), jnp.float32)
    for sub in range(4):
        sx = sub % 2
        sy = sub // 2
        tx = jnp.maximum(0.0, 1.0 - jnp.abs(px - (x0 + float(sx))))
        ty = jnp.maximum(0.0, 1.0 - jnp.abs(py - (y0 + float(sy))))
        perm = p_ref[sub]
        w128 = w128 + jnp.dot(aw * tx * ty, perm,
                              preferred_element_type=jnp.float32)
        x128 = x128 + jnp.dot(x0 + float(sx), perm,
                              preferred_element_type=jnp.float32)
        y128 = y128 + jnp.dot(y0 + float(sy), perm,
                              preferred_element_type=jnp.float32)
    wt_ref[0] = w128
    head128 = lax.broadcasted_iota(jnp.int32, (NQ, NH * NTERM), 1) // NTERM
    pos = y128.astype(jnp.int32) * W_ + x128.astype(jnp.int32)
    idx_ref[0] = ((head128 // 4) * (BS * HW * 4)
                  + (b * HW + pos) * 4 + head128 % 4)


def _query_side(query, ref_pts, wx, bx, wy, by, wa, ba, g, perms):
    n128 = NH * NP * 4
    return pl.pallas_call(
        _query_side_body,
        grid=(BS,),
        in_specs=[
            pl.BlockSpec((1, NQ, EMBED), lambda b: (b, 0, 0)),
            pl.BlockSpec((1, NQ, 2), lambda b: (b, 0, 0)),
            pl.BlockSpec((NH * NP, EMBED), lambda b: (0, 0)),
            pl.BlockSpec((NH * NP,), lambda b: (0,)),
            pl.BlockSpec((NH * NP, EMBED), lambda b: (0, 0)),
            pl.BlockSpec((NH * NP,), lambda b: (0,)),
            pl.BlockSpec((NH * NP, EMBED), lambda b: (0, 0)),
            pl.BlockSpec((NH * NP,), lambda b: (0,)),
            pl.BlockSpec((NH * NP, NH * NP), lambda b: (0, 0)),
            pl.BlockSpec((4, NH * NP, n128), lambda b: (0, 0, 0)),
        ],
        out_specs=[
            pl.BlockSpec((1, NQ, n128), lambda b: (b, 0, 0)),
            pl.BlockSpec((1, NQ, n128), lambda b: (b, 0, 0)),
        ],
        out_shape=[
            jax.ShapeDtypeStruct((BS, NQ, n128), jnp.int32),
            jax.ShapeDtypeStruct((BS, NQ, n128), jnp.float32),
        ],
    )(query, ref_pts, wx, bx, wy, by, wa, ba, g, perms)


def _sc_gather_reduce(vt_flat, idxr, wtr):
    mesh = plsc.VectorSubcoreMesh(core_axis_name="c", subcore_axis_name="s")

    @functools.partial(
        pl.kernel,
        mesh=mesh,
        compiler_params=pltpu.CompilerParams(use_tc_tiling_on_sc=False),
        out_type=jax.ShapeDtypeStruct((2, BS * NQ, 128), jnp.float32),
        scratch_types=[
            pltpu.VMEM((2, CH, NTERM), jnp.int32),
            pltpu.VMEM((2, CH * NTERM), jnp.int32),
            pltpu.VMEM((2, CH * NTERM, HD), jnp.float32),
            pltpu.VMEM((2, CH, NTERM), jnp.float32),
            pltpu.VMEM((CH, HD), jnp.float32),
            pltpu.SemaphoreType.DMA,
            pltpu.SemaphoreType.DMA,
        ],
    )
    def body(vt_hbm, idx_hbm, wt_hbm, out_hbm,
             idx_v, flat_v, rows_v, wt_v, out_v, sem0, sem1):
        w = lax.axis_index("s") * 2 + lax.axis_index("c")
        b = w // NH
        h = w % NH
        sems = [sem0, sem1]

        def fetch(q0, gl, ib):
            pltpu.sync_copy(
                idx_hbm.at[b, pl.ds(q0, gl), pl.ds(h * NTERM, NTERM)],
                idx_v.at[ib, pl.ds(0, gl)])
            pltpu.sync_copy(
                wt_hbm.at[b, pl.ds(q0, gl), pl.ds(h * NTERM, NTERM)],
                wt_v.at[ib, pl.ds(0, gl)])

            def repack(qi, c2):
                flat_v[ib, pl.ds(qi * NTERM, NTERM)] = idx_v[ib, qi, :]
                return c2

            lax.fori_loop(0, gl, repack, 0)
            handles = []
            for j in range(gl * NTERM // 128):
                handles.append(pltpu.async_copy(
                    vt_hbm.at[flat_v.at[ib, pl.ds(j * 128, 128)]],
                    rows_v.at[ib, pl.ds(j * 128, 128)], sems[ib]))
            return handles

        def compute(q0, ql, ib):
            def qstep(qi, c2):
                wrow = wt_v[ib, qi, :]
                acc0 = jnp.zeros((16,), jnp.float32)
                acc1 = jnp.zeros((16,), jnp.float32)
                for t in range(NTERM):
                    wsc = wrow[t]
                    acc0 = acc0 + wsc * rows_v[ib, qi * NTERM + t,
                                               pl.ds(0, 16)]
                    acc1 = acc1 + wsc * rows_v[ib, qi * NTERM + t,
                                               pl.ds(16, 16)]
                out_v[qi, pl.ds(0, 16)] = acc0
                out_v[qi, pl.ds(16, 16)] = acc1
                return c2

            lax.fori_loop(0, ql, qstep, 0)
            pltpu.sync_copy(
                out_v.at[pl.ds(0, ql)],
                out_hbm.at[h // 4, pl.ds(b * NQ + q0, ql),
                           pl.ds((h % 4) * HD, HD)])

        handles = fetch(SC_CHUNKS[0][0], SC_CHUNKS[0][1], 0)
        for i, (q0, gl, ql) in enumerate(SC_CHUNKS):
            for hnd in handles:
                hnd.wait()
            if i + 1 < len(SC_CHUNKS):
                nq0, ngl, _ = SC_CHUNKS[i + 1]
                handles = fetch(nq0, ngl, (i + 1) % 2)
            compute(q0, ql, i % 2)

    return body(vt_flat, idxr, wtr)


CQ = 1200


def _out_proj_body(s_ref, wt_ref, b_ref, o_ref):
    s = jnp.concatenate([s_ref[0], s_ref[1]], axis=1)
    acc = jnp.dot(s, wt_ref[...], preferred_element_type=jnp.float32)
    o_ref[...] = acc + b_ref[...][None, :]


def _out_proj(sampled2, w_out_t, b_out):
    return pl.pallas_call(
        _out_proj_body,
        grid=(BS * NQ // CQ,),
        in_specs=[
            pl.BlockSpec((2, CQ, 128), lambda i: (0, i, 0)),
            pl.BlockSpec((EMBED, EMBED), lambda i: (0, 0)),
            pl.BlockSpec((EMBED,), lambda i: (0,)),
        ],
        out_specs=pl.BlockSpec((CQ, EMBED), lambda i: (i, 0)),
        out_shape=jax.ShapeDtypeStruct((BS * NQ, EMBED), jnp.float32),
    )(sampled2, w_out_t, b_out)


def kernel(query, value, reference_points, spatial_shapes,
           W_off, b_off, W_attn, b_attn, W_val, b_val, W_out, b_out):
    f32 = jnp.float32
    j32 = jnp.arange(NH * NP)
    wx = W_off[j32 * 2].astype(f32)
    bx = b_off[j32 * 2].astype(f32)
    wy = W_off[j32 * 2 + 1].astype(f32)
    by = b_off[j32 * 2 + 1].astype(f32)
    wa = W_attn.astype(f32)
    ba = b_attn.astype(f32)
    head32a = j32 // NP
    g = (head32a[:, None] == head32a[None, :]).astype(f32)
    s4 = jnp.arange(4)[:, None, None]
    j128b = jnp.arange(NH * NTERM)[None, None, :]
    c32 = j32[None, :, None]
    perms = (j128b == (c32 // NP) * NTERM + s4 * NP + c32 % NP).astype(f32)

    vproj = _value_proj(value, W_val.T.astype(f32), b_val.astype(f32))
    vt_flat = vproj.reshape(BS * HW * NH, HD)

    ref_pts = reference_points[:, :, 0, :]
    idx, wt = _query_side(query, ref_pts, wx, bx, wy, by, wa, ba, g, perms)

    idxr = jnp.pad(idx, ((0, 0), (0, 4), (0, 0)))
    wtr = jnp.pad(wt, ((0, 0), (0, 4), (0, 0)))

    sampled = _sc_gather_reduce(vt_flat, idxr, wtr)

    out2d = _out_proj(sampled, W_out.T.astype(f32), b_out.astype(f32))
    return out2d.reshape(BS, NQ, EMBED)
